# baseline (speedup 1.0000x reference)
import functools

import jax
import jax.numpy as jnp
from jax import lax
from jax.experimental import pallas as pl
from jax.experimental.pallas import tpu as pltpu
from jax.experimental.pallas import tpu_sc as plsc

B = 16384
D = 32
NC = 2
NS = 16
NW = NC * NS
BPW = B // NW
NCHUNK = 4
CH = BPW // NCHUNK


def _body(gidx, aidx, oidx, zidx, Wg, Wa, Wo, Wz, out,
          gi_v, ai_v, oi_v, zi_v, g_v, a_v, o_v, z_v, sem):
    wid = lax.axis_index("s") * NC + lax.axis_index("c")
    base = wid * BPW
    pltpu.sync_copy(gidx.at[wid], gi_v)
    pltpu.sync_copy(aidx.at[wid], ai_v)
    pltpu.sync_copy(oidx.at[wid], oi_v)
    pltpu.sync_copy(zidx.at[wid], zi_v)
    copies = []
    for table, idx_v, rows_v in ((Wg, gi_v, g_v), (Wa, ai_v, a_v),
                                 (Wo, oi_v, o_v), (Wz, zi_v, z_v)):
        for j in range(NCHUNK):
            copies.append(pltpu.async_copy(
                table.at[idx_v.at[j]], rows_v.at[pl.ds(j * CH, CH)], sem))
    for c in copies:
        c.wait()
    pltpu.sync_copy(g_v, out.at[pl.ds(base, BPW), pl.ds(0 * D, D)])
    pltpu.sync_copy(a_v, out.at[pl.ds(base, BPW), pl.ds(1 * D, D)])
    pltpu.sync_copy(o_v, out.at[pl.ds(base, BPW), pl.ds(2 * D, D)])
    pltpu.sync_copy(z_v, out.at[pl.ds(base, BPW), pl.ds(3 * D, D)])


@jax.jit
def _lookup_concat(gidx, aidx, oidx, zidx, Wg, Wa, Wo, Wz):
    mesh = plsc.VectorSubcoreMesh(core_axis_name="c", subcore_axis_name="s",
                                  num_cores=NC, num_subcores=NS)
    f = pl.kernel(
        _body, mesh=mesh,
        out_type=jax.ShapeDtypeStruct((B, 4 * D), jnp.float32),
        scratch_types=[
            pltpu.VMEM((NCHUNK, CH), jnp.int32),
            pltpu.VMEM((NCHUNK, CH), jnp.int32),
            pltpu.VMEM((NCHUNK, CH), jnp.int32),
            pltpu.VMEM((NCHUNK, CH), jnp.int32),
            pltpu.VMEM((BPW, D), jnp.float32),
            pltpu.VMEM((BPW, D), jnp.float32),
            pltpu.VMEM((BPW, D), jnp.float32),
            pltpu.VMEM((BPW, D), jnp.float32),
            pltpu.SemaphoreType.DMA,
        ],
        compiler_params=pltpu.CompilerParams(use_tc_tiling_on_sc=False),
    )
    return f(gidx, aidx, oidx, zidx, Wg, Wa, Wo, Wz)


def kernel(gender_idx, age_idx, occupation_idx, area_idx,
           W_gender, W_age, W_occ, W_area):
    shp = (NW, NCHUNK, CH)
    return _lookup_concat(
        gender_idx.astype(jnp.int32).reshape(shp),
        age_idx.astype(jnp.int32).reshape(shp),
        occupation_idx.astype(jnp.int32).reshape(shp),
        area_idx.astype(jnp.int32).reshape(shp),
        W_gender, W_age, W_occ, W_area)
